# baseline (speedup 1.0000x reference)
import functools

import jax
import jax.numpy as jnp
from jax import lax
from jax.experimental import pallas as pl
from jax.experimental.pallas import tpu as pltpu
from jax.experimental.pallas import tpu_sc as plsc

NUM_CORES = 2
NUM_SUBCORES = 16
NUM_WORKERS = NUM_CORES * NUM_SUBCORES
CHUNK = 48
NBUF = 4


def kernel(x):
    B, C, H, W = x.shape
    HO = WO = 128
    SH, SW = H // HO, W // WO
    NR = B * C * HO
    per_w = NR // NUM_WORKERS
    n_chunks = per_w // CHUNK
    imgs_per_w = (B * C) // NUM_WORKERS
    chunks_per_img = HO // CHUNK

    x2 = x.reshape(B * C * H, W)

    mesh = plsc.VectorSubcoreMesh(
        core_axis_name="c",
        subcore_axis_name="s",
        num_cores=NUM_CORES,
        num_subcores=NUM_SUBCORES,
    )

    @functools.partial(
        pl.kernel,
        mesh=mesh,
        compiler_params=pltpu.CompilerParams(needs_layout_passes=False),
        out_type=jax.ShapeDtypeStruct((NR, WO), jnp.float32),
        scratch_types=[
            pltpu.VMEM((NBUF * CHUNK,), jnp.int32),
            pltpu.VMEM((NBUF * CHUNK, W), jnp.float32),
            pltpu.VMEM((NBUF * CHUNK, WO), jnp.float32),
            pltpu.SemaphoreType.DMA((NBUF,)),
            pltpu.SemaphoreType.DMA((NBUF,)),
        ],
    )
    def run(x_hbm, out_hbm, idx_v, buf, obuf, sem_in, sem_out):
        wid = lax.axis_index("s") * NUM_CORES + lax.axis_index("c")
        base = wid * per_w
        lanes = lax.iota(jnp.int32, 16)

        def start_gather(c, slot):
            wbase = jnp.full((16,), wid * imgs_per_w * H, jnp.int32)
            for u in range(CHUNK // 16):
                g = lanes + (c * CHUNK + u * 16)
                img = g >> 7
                rem = g & 127
                idx_v[pl.ds(slot * CHUNK + u * 16, 16)] = (
                    wbase + img * H + rem * SH
                )
            pltpu.make_async_copy(
                x_hbm.at[idx_v.at[pl.ds(slot * CHUNK, CHUNK)]],
                buf.at[pl.ds(slot * CHUNK, CHUNK)],
                sem_in.at[slot],
            ).start()

        def wait_gather(c, slot):
            pltpu.make_async_copy(
                x_hbm.at[idx_v.at[pl.ds(slot * CHUNK, CHUNK)]],
                buf.at[pl.ds(slot * CHUNK, CHUNK)],
                sem_in.at[slot],
            ).wait()

        def out_copy(c, slot):
            return pltpu.make_async_copy(
                obuf.at[pl.ds(slot * CHUNK, CHUNK)],
                out_hbm.at[pl.ds(base + c * CHUNK, CHUNK)],
                sem_out.at[slot],
            )

        def decimate(slot):
            @plsc.parallel_loop(0, CHUNK, unroll=4)
            def decimate_row(r):
                row = jnp.full((16,), slot * CHUNK, jnp.int32) + r
                for k in range(WO // 16):
                    cols = lanes * SW + (k * 16 * SW)
                    v = plsc.load_gather(buf, [row, cols])
                    obuf[slot * CHUNK + r, pl.ds(k * 16, 16)] = v

        for p in range(NBUF - 1):
            start_gather(p, p)

        def body(c, _):
            slot = lax.rem(c, NBUF)

            @pl.when(c + NBUF - 1 < n_chunks)
            def _():
                start_gather(c + NBUF - 1, lax.rem(c + NBUF - 1, NBUF))

            wait_gather(c, slot)

            @pl.when(c >= NBUF)
            def _():
                out_copy(c - NBUF, slot).wait()

            decimate(slot)
            out_copy(c, slot).start()
            return 0

        lax.fori_loop(0, n_chunks, body, 0, unroll=False)
        for t in range(NBUF):
            c = n_chunks - NBUF + t
            out_copy(c, c % NBUF).wait()

    out = run(x2)
    return out.reshape(B, C, HO, WO)


# speedup vs baseline: 1.0211x; 1.0211x over previous
import functools

import jax
import jax.numpy as jnp
from jax import lax
from jax.experimental import pallas as pl
from jax.experimental.pallas import tpu as pltpu
from jax.experimental.pallas import tpu_sc as plsc

NUM_CORES = 2
NUM_SUBCORES = 16
NUM_WORKERS = NUM_CORES * NUM_SUBCORES
CHUNK = 48
NBUF = 4


def kernel(x):
    B, C, H, W = x.shape
    HO = WO = 128
    SH, SW = H // HO, W // WO
    NR = B * C * HO
    per_w = NR // NUM_WORKERS
    n_chunks = per_w // CHUNK
    imgs_per_w = (B * C) // NUM_WORKERS
    chunks_per_img = HO // CHUNK

    x2 = x.reshape(B * C * H, W)

    mesh = plsc.VectorSubcoreMesh(
        core_axis_name="c",
        subcore_axis_name="s",
        num_cores=NUM_CORES,
        num_subcores=NUM_SUBCORES,
    )

    @functools.partial(
        pl.kernel,
        mesh=mesh,
        compiler_params=pltpu.CompilerParams(needs_layout_passes=False),
        out_type=jax.ShapeDtypeStruct((NR, WO), jnp.float32),
        scratch_types=[
            pltpu.VMEM((NBUF * CHUNK,), jnp.int32),
            pltpu.VMEM((NBUF * CHUNK, W), jnp.float32),
            pltpu.VMEM((NBUF * CHUNK, WO), jnp.float32),
            pltpu.VMEM_SHARED((2 * CHUNK, WO), jnp.float32),
            pltpu.SemaphoreType.DMA((NBUF,)),
            pltpu.SemaphoreType.DMA((2,)),
        ],
    )
    def run(x_hbm, out_hbm, idx_v, buf, obuf, spm, sem_in, sem_out):
        sid = lax.axis_index("s")
        wid = lax.axis_index("s") * NUM_CORES + lax.axis_index("c")
        base = wid * per_w
        lanes = lax.iota(jnp.int32, 16)

        def start_gather(c, slot):
            wbase = jnp.full((16,), wid * imgs_per_w * H, jnp.int32)
            for u in range(CHUNK // 16):
                g = lanes + (c * CHUNK + u * 16)
                img = g >> 7
                rem = g & 127
                idx_v[pl.ds(slot * CHUNK + u * 16, 16)] = (
                    wbase + img * H + rem * SH
                )
            pltpu.make_async_copy(
                x_hbm.at[idx_v.at[pl.ds(slot * CHUNK, CHUNK)]],
                buf.at[pl.ds(slot * CHUNK, CHUNK)],
                sem_in.at[slot],
            ).start()

        def wait_gather(c, slot):
            pltpu.make_async_copy(
                x_hbm.at[idx_v.at[pl.ds(slot * CHUNK, CHUNK)]],
                buf.at[pl.ds(slot * CHUNK, CHUNK)],
                sem_in.at[slot],
            ).wait()

        def out_copy(c, oslot):
            return pltpu.make_async_copy(
                spm.at[pl.ds(oslot * CHUNK, CHUNK)],
                out_hbm.at[pl.ds(base + c * CHUNK, CHUNK)],
                sem_out.at[oslot],
            )

        def decimate(slot):
            @plsc.parallel_loop(0, CHUNK, unroll=4)
            def decimate_row(r):
                row = jnp.full((16,), slot * CHUNK, jnp.int32) + r
                for k in range(WO // 16):
                    cols = lanes * SW + (k * 16 * SW)
                    v = plsc.load_gather(buf, [row, cols])
                    obuf[slot * CHUNK + r, pl.ds(k * 16, 16)] = v

        for p in range(NBUF - 1):
            start_gather(p, p)

        def body(c, _):
            slot = lax.rem(c, NBUF)

            @pl.when(c + NBUF - 1 < n_chunks)
            def _():
                start_gather(c + NBUF - 1, lax.rem(c + NBUF - 1, NBUF))

            wait_gather(c, slot)
            oslot = lax.rem(c, 2)

            @pl.when(c >= 2)
            def _():
                out_copy(c - 2, oslot).wait()

            decimate(slot)
            pltpu.sync_copy(
                obuf.at[pl.ds(slot * CHUNK, CHUNK)],
                spm.at[pl.ds(oslot * CHUNK, CHUNK)],
            )
            out_copy(c, oslot).start()
            return 0

        lax.fori_loop(0, n_chunks, body, 0, unroll=False)
        out_copy(n_chunks - 2, n_chunks % 2).wait()
        out_copy(n_chunks - 1, (n_chunks - 1) % 2).wait()

    out = run(x2)
    return out.reshape(B, C, HO, WO)
